# baseline (speedup 1.0000x reference)
import functools

import jax
import jax.numpy as jnp
from jax.experimental import pallas as pl
from jax.experimental.pallas import tpu as pltpu

_ALPHA_GAIN = (1.0 / (0.01 * 1000000.0)) / 2.0

_CHUNK_ROWS = 512
_K = 6


def _body(x_hbm, o_hbm, in_buf, out_buf, in_sem, out_sem, *, nch):
    def start_in(i, s):
        pltpu.make_async_copy(
            x_hbm.at[pl.ds(i * _CHUNK_ROWS, _CHUNK_ROWS), :],
            in_buf.at[s], in_sem.at[s]).start()

    def wait_in(i, s):
        pltpu.make_async_copy(
            x_hbm.at[pl.ds(i * _CHUNK_ROWS, _CHUNK_ROWS), :],
            in_buf.at[s], in_sem.at[s]).wait()

    def start_out(i, s):
        pltpu.make_async_copy(
            out_buf.at[s],
            o_hbm.at[pl.ds(i * _CHUNK_ROWS, _CHUNK_ROWS), :],
            out_sem.at[s]).start()

    def wait_out(i, s):
        pltpu.make_async_copy(
            out_buf.at[s],
            o_hbm.at[pl.ds(i * _CHUNK_ROWS, _CHUNK_ROWS), :],
            out_sem.at[s]).wait()

    for i in range(min(_K, nch)):
        start_in(i, i)
    for i in range(nch):
        s = i % _K
        wait_in(i, s)
        if i >= _K:
            wait_out(i - _K, s)
        out_buf[s] = in_buf[s] * _ALPHA_GAIN
        start_out(i, s)
        if i + _K < nch:
            start_in(i + _K, s)
    for i in range(max(nch - _K, 0), nch):
        wait_out(i, i % _K)


def kernel(t_in, rate_hopping, y_in, inds_surf, inds_mant, dy_surf_gain, dy_surf_loss, inds_r_m2s):
    b, n = dy_surf_gain.shape
    nch = b // _CHUNK_ROWS
    return pl.pallas_call(
        functools.partial(_body, nch=nch),
        in_specs=[pl.BlockSpec(memory_space=pltpu.MemorySpace.HBM)],
        out_specs=pl.BlockSpec(memory_space=pltpu.MemorySpace.HBM),
        out_shape=jax.ShapeDtypeStruct((b, n), dy_surf_gain.dtype),
        scratch_shapes=[
            pltpu.MemorySpace.VMEM((_K, _CHUNK_ROWS, n), jnp.float32),
            pltpu.MemorySpace.VMEM((_K, _CHUNK_ROWS, n), jnp.float32),
            pltpu.SemaphoreType.DMA((_K,)),
            pltpu.SemaphoreType.DMA((_K,)),
        ],
    )(dy_surf_gain)
